# baseline (speedup 1.0000x reference)
import jax
import jax.numpy as jnp
from jax import lax
from jax.experimental import pallas as pl
from jax.experimental.pallas import tpu as pltpu

N_FACTORS = 64
HIDDEN_1 = 256
BATCH = 16384
BLK = 512
NBLK = BATCH // BLK
GRAN = 16
NGRAN = 1000000 // GRAN


def _body(uid_s, iid_s, uoff, ioff, utab, itab, w1u_t, w1i_t, b1r, w2t, b2s,
          o_ref, xu, xi, su, si):
    i = pl.program_id(0)
    base = i * BLK

    def issue(j, c):
        u = uid_s[base + j]
        v = iid_s[base + j]
        pltpu.make_async_copy(
            utab.at[pl.ds(u // GRAN, 1)], xu.at[pl.ds(j, 1)], su).start()
        pltpu.make_async_copy(
            itab.at[pl.ds(v // GRAN, 1)], xi.at[pl.ds(j, 1)], si).start()
        return c

    lax.fori_loop(0, BLK, issue, 0, unroll=8)
    pltpu.make_async_copy(utab.at[pl.ds(0, BLK)], xu, su).wait()
    pltpu.make_async_copy(itab.at[pl.ds(0, BLK)], xi, si).wait()

    def extract(panel, off_ref):
        off = off_ref[...] % GRAN
        sub = off // 2
        half = (off % 2).astype(jnp.float32)
        sel = (sub == lax.broadcasted_iota(jnp.int32, (BLK, 8), 1))
        t = jnp.sum(panel[...] * sel.astype(jnp.float32)[:, :, None], axis=1)
        return t[:, :N_FACTORS] * (1.0 - half) + t[:, N_FACTORS:] * half

    xum = extract(xu, uoff)
    xim = extract(xi, ioff)
    h = jnp.dot(xum, w1u_t[...], preferred_element_type=jnp.float32)
    h = h + jnp.dot(xim, w1i_t[...], preferred_element_type=jnp.float32)
    h = jnp.maximum(h + b1r[...], 0.0)
    o = jnp.dot(h, w2t[...], preferred_element_type=jnp.float32)
    o_ref[...] = o + b2s[0, 0]


@jax.jit
def _fused(uid, iid, uoff, ioff, utab3, itab3, w1u_t, w1i_t, b1r, w2t, b2s):
    return pl.pallas_call(
        _body,
        grid=(NBLK,),
        in_specs=[
            pl.BlockSpec(memory_space=pltpu.SMEM),
            pl.BlockSpec(memory_space=pltpu.SMEM),
            pl.BlockSpec((BLK, 1), lambda i: (i, 0)),
            pl.BlockSpec((BLK, 1), lambda i: (i, 0)),
            pl.BlockSpec(memory_space=pl.ANY),
            pl.BlockSpec(memory_space=pl.ANY),
            pl.BlockSpec((N_FACTORS, HIDDEN_1), lambda i: (0, 0)),
            pl.BlockSpec((N_FACTORS, HIDDEN_1), lambda i: (0, 0)),
            pl.BlockSpec((1, HIDDEN_1), lambda i: (0, 0)),
            pl.BlockSpec((HIDDEN_1, 1), lambda i: (0, 0)),
            pl.BlockSpec((1, 1), lambda i: (0, 0), memory_space=pltpu.SMEM),
        ],
        out_specs=pl.BlockSpec((BLK, 1), lambda i: (i, 0)),
        out_shape=jax.ShapeDtypeStruct((BATCH, 1), jnp.float32),
        scratch_shapes=[
            pltpu.VMEM((BLK, 8, 128), jnp.float32),
            pltpu.VMEM((BLK, 8, 128), jnp.float32),
            pltpu.SemaphoreType.DMA,
            pltpu.SemaphoreType.DMA,
        ],
        compiler_params=pltpu.CompilerParams(
            dimension_semantics=("arbitrary",)),
    )(uid, iid, uoff, ioff, utab3, itab3, w1u_t, w1i_t, b1r, w2t, b2s)


def kernel(user_ids, item_ids, user_table, item_table, W1, b1, W2, b2):
    uid = user_ids.astype(jnp.int32)
    iid = item_ids.astype(jnp.int32)
    out = _fused(uid, iid, uid[:, None], iid[:, None],
                 user_table.reshape(NGRAN, 8, 128),
                 item_table.reshape(NGRAN, 8, 128),
                 W1[:, :N_FACTORS].T, W1[:, N_FACTORS:].T,
                 b1[None, :], W2.T, b2[None, :])
    return out.reshape(BATCH)
